# Initial kernel scaffold; baseline (speedup 1.0000x reference)
import jax
import jax.numpy as jnp
from jax.experimental import pallas as pl


def kernel(center_table, context_table, word_semantics, names, center_input, context_output, negative_samples):
    raise NotImplementedError("write your pallas kernel here")


# speedup vs baseline: 5.0932x; 5.0932x over previous
import functools

import jax
import jax.numpy as jnp
from jax import lax
from jax.experimental import pallas as pl
from jax.experimental.pallas import tpu as pltpu
from jax.experimental.pallas import tpu_sc as plsc

NC = 2
NS = 16
NW = NC * NS
L = 16


def _sc_scores(ctab, otab, ws, cidx, oidx, nidx, B, D, NEG, S):
    BPW = B // NW
    NSB = BPW // S
    DJ = D // L
    mesh = plsc.VectorSubcoreMesh(core_axis_name="c", subcore_axis_name="s")

    @functools.partial(
        pl.kernel,
        out_type=[jax.ShapeDtypeStruct((B,), jnp.float32)] * 3,
        mesh=mesh,
        compiler_params=pltpu.CompilerParams(needs_layout_passes=False,
                                             use_tc_tiling_on_sc=False),
        scratch_types=[
            pltpu.VMEM((NSB, S), jnp.int32),
            pltpu.VMEM((NSB, S), jnp.int32),
            pltpu.VMEM((NEG, NSB, S), jnp.int32),
            pltpu.VMEM((D,), jnp.float32),
            pltpu.VMEM((S, D), jnp.float32),
            pltpu.VMEM((S, D), jnp.float32),
            pltpu.VMEM((NEG, S, D), jnp.float32),
            pltpu.VMEM((BPW,), jnp.float32),
            pltpu.VMEM((BPW,), jnp.float32),
            pltpu.VMEM((BPW,), jnp.float32),
            pltpu.SemaphoreType.DMA,
        ],
    )
    def k(ctab_h, otab_h, ws_h, cidx_h, oidx_h, nidx_h,
          pos_o, negd_o, sent_o,
          cidx_v, oidx_v, nidx_v, ws_v, v_v, u_v, nbuf,
          sc_pos, sc_neg, sc_sent, sem):
        wid = lax.axis_index("s") * NC + lax.axis_index("c")
        base = wid * BPW
        pltpu.sync_copy(cidx_h.at[wid], cidx_v)
        pltpu.sync_copy(oidx_h.at[wid], oidx_v)
        pltpu.sync_copy(nidx_h.at[wid], nidx_v)
        pltpu.sync_copy(ws_h, ws_v)
        lane = lax.iota(jnp.int32, L)

        @pl.loop(0, NSB)
        def _sb(sb):
            cps = [pltpu.async_copy(ctab_h.at[cidx_v.at[sb]], v_v, sem),
                   pltpu.async_copy(otab_h.at[oidx_v.at[sb]], u_v, sem)]
            for n in range(NEG):
                cps.append(
                    pltpu.async_copy(otab_h.at[nidx_v.at[n, sb]],
                                     nbuf.at[n], sem))
            for cp in cps:
                cp.wait()
            wsr = [ws_v[pl.ds(L * j, L)] for j in range(DJ)]

            @pl.loop(0, S // L)
            def _g(g):
                su = jnp.zeros((L,), jnp.float32)
                sn = jnp.zeros((L,), jnp.float32)
                ss = jnp.zeros((L,), jnp.float32)
                for r in range(L):
                    row = g * L + r
                    vr = [v_v[row, pl.ds(L * j, L)] for j in range(DJ)]
                    au = vr[0] * u_v[row, pl.ds(0, L)]
                    asn = vr[0] * wsr[0]
                    for j in range(1, DJ):
                        au = au + vr[j] * u_v[row, pl.ds(L * j, L)]
                        asn = asn + vr[j] * wsr[j]
                    an = vr[0] * nbuf[0, row, pl.ds(0, L)]
                    for j in range(1, DJ):
                        an = an + vr[j] * nbuf[0, row, pl.ds(L * j, L)]
                    for n in range(1, NEG):
                        for j in range(DJ):
                            an = an + vr[j] * nbuf[n, row, pl.ds(L * j, L)]
                    here = lane == r
                    su = jnp.where(here, jnp.sum(au), su)
                    sn = jnp.where(here, jnp.sum(an), sn)
                    ss = jnp.where(here, jnp.sum(asn), ss)
                off = sb * S + g * L
                sc_pos[pl.ds(off, L)] = su
                sc_neg[pl.ds(off, L)] = sn
                sc_sent[pl.ds(off, L)] = ss

        pltpu.sync_copy(sc_pos, pos_o.at[pl.ds(base, BPW)])
        pltpu.sync_copy(sc_neg, negd_o.at[pl.ds(base, BPW)])
        pltpu.sync_copy(sc_sent, sent_o.at[pl.ds(base, BPW)])

    return k(ctab, otab, ws, cidx, oidx, nidx)


def _tc_loss(pos, negd, sent, B):
    R = 128
    C = B // R

    def body(pos_ref, neg_ref, sent_ref, out_ref):
        p = pos_ref[...]
        nd = neg_ref[...]
        st = sent_ref[...]

        def logsig(x):
            return jnp.minimum(x, 0.0) - jnp.log1p(jnp.exp(-jnp.abs(x)))

        sig = 1.0 / (1.0 + jnp.exp(-st))
        val = logsig(p) + logsig(-nd) - jnp.abs(sig - 0.5)
        out_ref[0, 0] = -jnp.sum(val) / B

    out = pl.pallas_call(
        body,
        out_shape=jax.ShapeDtypeStruct((1, 1), jnp.float32),
        out_specs=pl.BlockSpec(memory_space=pltpu.SMEM),
    )(pos.reshape(R, C), negd.reshape(R, C), sent.reshape(R, C))
    return out[0, 0]


def kernel(center_table, context_table, word_semantics, names,
           center_input, context_output, negative_samples):
    B = center_input.shape[0]
    D = center_table.shape[1]
    NEG = negative_samples.shape[1]
    S = 64
    BPW = B // NW
    NSB = BPW // S

    cidx = center_input.reshape(NW, NSB, S)
    oidx = context_output.reshape(NW, NSB, S)
    nidx = negative_samples.T.reshape(NEG, NW, NSB, S).transpose(1, 0, 2, 3)

    pos, negd, sent = _sc_scores(center_table, context_table, word_semantics,
                                 cidx, oidx, nidx, B, D, NEG, S)
    return _tc_loss(pos, negd, sent, B)
